# Initial kernel scaffold; baseline (speedup 1.0000x reference)
import jax
import jax.numpy as jnp
from jax.experimental import pallas as pl


def kernel(x, edge_index):
    raise NotImplementedError("write your pallas kernel here")


# speedup vs baseline: 3.3641x; 3.3641x over previous
import functools

import jax
import jax.numpy as jnp
from jax import lax
from jax.experimental import pallas as pl
from jax.experimental.pallas import tpu as pltpu
from jax.experimental.pallas import tpu_sc as plsc

_NC = 2
_NS = 16
_NW = _NC * _NS
_LANES = 16
_C = 80


def _make_edge_sub(n_nodes: int, d: int, n_edges: int):
    e_per_w = n_edges // _NW
    n_chunks = e_per_w // _C
    mesh = plsc.VectorSubcoreMesh(core_axis_name="c", subcore_axis_name="s")

    @functools.partial(
        pl.kernel,
        mesh=mesh,
        out_type=jax.ShapeDtypeStruct((n_edges, d), jnp.float32),
        scratch_types=[
            pltpu.VMEM((_C,), jnp.int32),
            pltpu.VMEM((_C,), jnp.int32),
            pltpu.VMEM((_C, d), jnp.float32),
            pltpu.VMEM((_C, d), jnp.float32),
            pltpu.SemaphoreType.DMA,
        ],
    )
    def edge_sub(x_hbm, src_hbm, dst_hbm, out_hbm, idx_s, idx_d, rows_s,
                 rows_d, sem):
        wid = lax.axis_index("s") * _NC + lax.axis_index("c")
        base_w = wid * e_per_w

        def chunk_body(ci, _):
            base = base_w + ci * _C
            pltpu.sync_copy(src_hbm.at[pl.ds(base, _C)], idx_s)
            pltpu.sync_copy(dst_hbm.at[pl.ds(base, _C)], idx_d)
            cp_s = pltpu.async_copy(x_hbm.at[idx_s], rows_s, sem)
            cp_d = pltpu.async_copy(x_hbm.at[idx_d], rows_d, sem)
            cp_s.wait()
            cp_d.wait()

            def sub_row(e, _):
                for j in range(d // _LANES):
                    sl = pl.ds(j * _LANES, _LANES)
                    rows_s[e, sl] = rows_s[e, sl] - rows_d[e, sl]
                return 0

            lax.fori_loop(0, _C, sub_row, 0)
            pltpu.sync_copy(rows_s, out_hbm.at[pl.ds(base, _C)])
            return 0

        lax.fori_loop(0, n_chunks, chunk_body, 0)

    return edge_sub


def kernel(x, edge_index):
    n_nodes, d = x.shape
    n_edges = edge_index.shape[1]
    ei = edge_index.astype(jnp.int32)
    fn = _make_edge_sub(n_nodes, d, n_edges)
    return fn(x, ei[0], ei[1])


# speedup vs baseline: 7.8008x; 2.3188x over previous
import functools

import jax
import jax.numpy as jnp
from jax import lax
from jax.experimental import pallas as pl
from jax.experimental.pallas import tpu as pltpu
from jax.experimental.pallas import tpu_sc as plsc

_NC = 2
_NS = 16
_NW = _NC * _NS
_LANES = 16
_C = 40
_NBUF = 5


def _make_edge_sub(d: int, n_edges: int):
    e_per_w = n_edges // _NW
    n_chunks = e_per_w // _C
    n_groups = n_chunks // _NBUF
    mesh = plsc.VectorSubcoreMesh(core_axis_name="c", subcore_axis_name="s")

    @functools.partial(
        pl.kernel,
        mesh=mesh,
        out_type=jax.ShapeDtypeStruct((n_edges, d), jnp.float32),
        scratch_types=[
            pltpu.VMEM((e_per_w,), jnp.int32),
            pltpu.VMEM((e_per_w,), jnp.int32),
            pltpu.VMEM((_NBUF, _C, d), jnp.float32),
            pltpu.VMEM((_NBUF, _C, d), jnp.float32),
            pltpu.VMEM((_NBUF, _C, d), jnp.float32),
            pltpu.SemaphoreType.DMA((_NBUF,)),
            pltpu.SemaphoreType.DMA((_NBUF,)),
        ],
    )
    def edge_sub(x_hbm, src_hbm, dst_hbm, out_hbm, idx_s, idx_d, rows_s,
                 rows_d, obuf, gsem, wsem):
        wid = lax.axis_index("s") * _NC + lax.axis_index("c")
        base_w = wid * e_per_w

        def gather_cp(g, b):
            lo = g * _C
            cp_s = pltpu.make_async_copy(
                x_hbm.at[idx_s.at[pl.ds(lo, _C)]], rows_s.at[b], gsem.at[b])
            cp_d = pltpu.make_async_copy(
                x_hbm.at[idx_d.at[pl.ds(lo, _C)]], rows_d.at[b], gsem.at[b])
            return cp_s, cp_d

        def fire_gather(g, b):
            cp_s, cp_d = gather_cp(g, b)
            cp_s.start()
            cp_d.start()

        def wait_gather(g, b):
            cp_s, cp_d = gather_cp(g, b)
            cp_s.wait()
            cp_d.wait()

        def write_cp(g, b):
            return pltpu.make_async_copy(
                obuf.at[b], out_hbm.at[pl.ds(base_w + g * _C, _C)], wsem.at[b])

        def subtract(b):
            def sub_row(e, _):
                for j in range(d // _LANES):
                    sl = pl.ds(j * _LANES, _LANES)
                    obuf[b, e, sl] = rows_s[b, e, sl] - rows_d[b, e, sl]
                return 0

            lax.fori_loop(0, _C, sub_row, 0)

        pltpu.sync_copy(src_hbm.at[pl.ds(base_w, e_per_w)], idx_s)
        pltpu.sync_copy(dst_hbm.at[pl.ds(base_w, e_per_w)], idx_d)

        for b in range(_NBUF):
            fire_gather(b, b)

        for b in range(_NBUF):
            wait_gather(b, b)
            subtract(b)
            write_cp(b, b).start()
            fire_gather(b + _NBUF, b)

        def group(go, _):
            for b in range(_NBUF):
                g = go * _NBUF + b
                wait_gather(g, b)
                write_cp(g - _NBUF, b).wait()
                subtract(b)
                write_cp(g, b).start()
                fire_gather(g + _NBUF, b)
            return 0

        lax.fori_loop(1, n_groups - 1, group, 0)

        for b in range(_NBUF):
            g = (n_groups - 1) * _NBUF + b
            wait_gather(g, b)
            write_cp(g - _NBUF, b).wait()
            subtract(b)
            write_cp(g, b).start()

        for b in range(_NBUF):
            g = (n_groups - 1) * _NBUF + b
            write_cp(g, b).wait()

    return edge_sub


def kernel(x, edge_index):
    d = x.shape[1]
    n_edges = edge_index.shape[1]
    ei = edge_index.astype(jnp.int32)
    fn = _make_edge_sub(d, n_edges)
    return fn(x, ei[0], ei[1])
